# baseline (speedup 1.0000x reference)
import math
from functools import partial

import jax
import jax.numpy as jnp
from jax import lax
from jax.experimental import pallas as pl
from jax.experimental.pallas import tpu as pltpu

_EPS = 1e-5


def _band_generic(w_hwio, width, row_of, col_of, rows, cols, dtype):
    KH, KW, Cin, Cout = w_hwio.shape
    pad = KH // 2
    r = jnp.arange(rows)
    q = jnp.arange(cols)
    c, wp = row_of(r)
    o, w = col_of(q)
    kw = wp[:, None] - w[None, :] + pad
    valid = (kw >= 0) & (kw < KW)
    kwc = jnp.clip(kw, 0, KW - 1)
    kh = jnp.arange(KH)[:, None, None]
    b = w_hwio[kh, kwc[None], c[:, None][None], o[None, :][None]]
    return jnp.where(valid[None], b, 0.0).astype(dtype)


def _conv_taps(xb, band_ref, height):
    M = xb.shape[0]
    KH = band_ref.shape[0]
    pad = KH // 2
    hmod = lax.broadcasted_iota(jnp.int32, xb.shape, 0) % height
    acc = None
    for kh in range(KH):
        d = kh - pad
        if d == 0:
            lhs = xb
        else:
            lhs = pltpu.roll(xb, (-d) % M, 0)
            valid = hmod < (height - d) if d > 0 else hmod >= (-d)
            lhs = jnp.where(valid, lhs, jnp.zeros_like(lhs))
        part = jnp.dot(lhs, band_ref[kh], preferred_element_type=jnp.float32)
        acc = part if acc is None else acc + part
    return acc


def _finish_bn(sacc, g, b, ch, cnt):
    tot = sacc[0:1, :]
    tsq = sacc[1:2, :]
    L = tot.shape[1]
    step = ch
    while step < L:
        tot = tot + pltpu.roll(tot, step, 1)
        tsq = tsq + pltpu.roll(tsq, step, 1)
        step *= 2
    mean = tot / cnt
    var = jnp.maximum(tsq / cnt - mean * mean, 0.0)
    scale = g * lax.rsqrt(var + _EPS)
    shift = b - mean * scale
    return scale, shift


def _accumulate(y, t, s_ref):
    part = jnp.concatenate([jnp.sum(y, 0, keepdims=True),
                            jnp.sum(y * y, 0, keepdims=True)], 0)

    @pl.when(t == 0)
    def _():
        s_ref[...] = part

    @pl.when((t > 0))
    def _():
        s_ref[...] = s_ref[...] + part


def _norm_chunk(acc_ref, s_ref, g_ref, b_ref, c_ref, j, mb, height, ch, cnt):
    scale, shift = _finish_bn(s_ref[...], g_ref[0], b_ref[0], ch, cnt)
    y = acc_ref[pl.ds(j * mb, mb), :]
    nb = mb // height
    l = y.shape[1]
    bias = jnp.broadcast_to(c_ref[0][:, None, :], (nb, height, l)).reshape(mb, l)
    return jnp.maximum(y * scale + shift, 0.0) + bias


def _k1_body(x_ref, band_ref, g_ref, b_ref, c_ref, z_ref, acc_ref, s_ref, *,
             height, t_steps, ch, cnt):
    t = pl.program_id(1)
    mb = x_ref.shape[0]

    @pl.when(t < t_steps)
    def _():
        y = _conv_taps(x_ref[...], band_ref, height)
        acc_ref[pl.ds(t * mb, mb), :] = y
        _accumulate(y, t, s_ref)

    @pl.when(t >= t_steps)
    def _():
        j = t - t_steps
        z = _norm_chunk(acc_ref, s_ref, g_ref, b_ref, c_ref, j, mb, height,
                        ch, cnt)
        z_ref[0] = z.astype(z_ref.dtype)


def _k2_body(za_ref, zb_ref, band_ref, g_ref, b_ref, c_ref, o_ref, acc_ref,
             s_ref, *, height, width, t_steps, ch, cnt):
    t = pl.program_id(1)
    mb = za_ref.shape[1]

    @pl.when(t < t_steps)
    def _():
        lhs = jnp.concatenate([za_ref[0], zb_ref[0]], axis=1)
        y = _conv_taps(lhs, band_ref, height)
        acc_ref[pl.ds(t * mb, mb), :] = y
        _accumulate(y, t, s_ref)

    @pl.when(t >= t_steps)
    def _():
        j = t - t_steps
        z = _norm_chunk(acc_ref, s_ref, g_ref, b_ref, c_ref, j, mb, height,
                        ch, cnt)
        nb = mb // height
        z = z.reshape(nb, height, width, z.shape[1] // width)
        o_ref[...] = z.transpose(0, 3, 1, 2)


def kernel(x_nchw, condition, w1_hwio, w2_hwio, bn1_gamma, bn1_beta,
           bn2_gamma, bn2_beta, emb1_w, emb1_b, emb2_w, emb2_b):
    N, Cin, H, W = x_nchw.shape
    KH, KW, _, Cout = w1_hwio.shape
    assert KH == KW and KH % 2 == 1
    assert W & (W - 1) == 0
    assert Cout % 2 == 0
    Ch = Cout // 2
    L = W * Ch
    WC_in = W * Cin
    M = N * H
    cnt = float(M * W)
    T = 16
    while (N % T) or (((N // T) * H) % 8):
        T //= 2
    Nb = N // T
    Mb = Nb * H

    x2 = jnp.transpose(x_nchw, (0, 2, 1, 3)).reshape(M, WC_in).astype(jnp.bfloat16)

    def col_half(q):
        half, rem = q // L, q % L
        w, oh = rem // Ch, rem % Ch
        return half * Ch + oh, w

    band1 = _band_generic(w1_hwio, W,
                          lambda r: (r // W, r % W),
                          col_half, W * Cin, W * Cout, jnp.bfloat16)
    band2 = _band_generic(w2_hwio, W, col_half,
                          col_half, W * Cout, W * Cout, jnp.bfloat16)

    def half_tile(v):
        v2 = v.reshape(2, Ch)
        return jnp.tile(v2[:, None, :], (1, 1, W)).astype(jnp.float32)

    g1t, b1t = half_tile(bn1_gamma), half_tile(bn1_beta)
    g2t, b2t = half_tile(bn2_gamma), half_tile(bn2_beta)

    def half_cond(emb_w, emb_b):
        c = (condition @ emb_w.T + emb_b).reshape(N, 2, Ch)
        c = jnp.transpose(c, (1, 0, 2))
        return jnp.tile(c, (1, 1, W)).astype(jnp.float32)

    c1t = half_cond(emb1_w, emb1_b)
    c2t = half_cond(emb2_w, emb2_b)

    cparams = pltpu.CompilerParams(
        dimension_semantics=("parallel", "arbitrary"))

    def rd(i):
        return jnp.minimum(i, T - 1)

    def wr(i):
        return jnp.maximum(i - T, 0)

    z1 = pl.pallas_call(
        partial(_k1_body, height=H, t_steps=T, ch=Ch, cnt=cnt),
        grid=(2, 2 * T),
        in_specs=[
            pl.BlockSpec((Mb, WC_in), lambda k, t: (rd(t), 0)),
            pl.BlockSpec((KH, WC_in, L), lambda k, t: (0, 0, k)),
            pl.BlockSpec((1, 1, L), lambda k, t: (k, 0, 0)),
            pl.BlockSpec((1, 1, L), lambda k, t: (k, 0, 0)),
            pl.BlockSpec((1, Nb, L), lambda k, t: (k, wr(t), 0)),
        ],
        out_specs=pl.BlockSpec((1, Mb, L), lambda k, t: (k, wr(t), 0)),
        out_shape=jax.ShapeDtypeStruct((2, M, L), jnp.bfloat16),
        scratch_shapes=[pltpu.VMEM((M, L), jnp.float32),
                        pltpu.VMEM((2, L), jnp.float32)],
        compiler_params=cparams,
    )(x2, band1, g1t, b1t, c1t)

    out = pl.pallas_call(
        partial(_k2_body, height=H, width=W, t_steps=T, ch=Ch, cnt=cnt),
        grid=(2, 2 * T),
        in_specs=[
            pl.BlockSpec((1, Mb, L), lambda k, t: (0, rd(t), 0)),
            pl.BlockSpec((1, Mb, L), lambda k, t: (1, rd(t), 0)),
            pl.BlockSpec((KH, W * Cout, L), lambda k, t: (0, 0, k)),
            pl.BlockSpec((1, 1, L), lambda k, t: (k, 0, 0)),
            pl.BlockSpec((1, 1, L), lambda k, t: (k, 0, 0)),
            pl.BlockSpec((1, Nb, L), lambda k, t: (k, wr(t), 0)),
        ],
        out_specs=pl.BlockSpec((Nb, Ch, H, W), lambda k, t: (wr(t), k, 0, 0)),
        out_shape=jax.ShapeDtypeStruct((N, Cout, H, W), jnp.float32),
        scratch_shapes=[pltpu.VMEM((M, L), jnp.float32),
                        pltpu.VMEM((2, L), jnp.float32)],
        compiler_params=cparams,
    )(z1, z1, band2, g2t, b2t, c2t)

    return out


# speedup vs baseline: 49.5978x; 49.5978x over previous
import math
from functools import partial

import jax
import jax.numpy as jnp
from jax import lax
from jax.experimental import pallas as pl
from jax.experimental.pallas import tpu as pltpu

_EPS = 1e-5


def _taps(w_hwio, width):
    KH, KW, Cin, Cout = w_hwio.shape
    pad = KH // 2
    idx = jnp.arange(width)
    rel = idx[:, None] - idx[None, :] + pad
    inband = (rel >= 0) & (rel < KW)
    t = w_hwio[:, jnp.clip(rel, 0, KW - 1)]
    return jnp.where(inband[None, :, :, None, None], t, 0.0)


def _band1_half(w_hwio, width, dtype):
    KH, KW, Cin, Cout = w_hwio.shape
    t = _taps(w_hwio, width).reshape(KH, width, width, Cin, 2, Cout // 2)
    t = jnp.transpose(t, (0, 1, 3, 4, 2, 5))
    return t.reshape(KH, width * Cin, width * Cout).astype(dtype)


def _band2_half(w_hwio, width, dtype):
    KH, KW, C2, Cout = w_hwio.shape
    ch = Cout // 2
    t = _taps(w_hwio, width).reshape(KH, width, width, 2, ch, 2, ch)
    t = jnp.transpose(t, (0, 3, 1, 4, 5, 2, 6))
    return t.reshape(KH, width * C2, width * Cout).astype(dtype)


def _conv_taps(xb, band_ref, height):
    M = xb.shape[0]
    KH = band_ref.shape[0]
    pad = KH // 2
    hmod = lax.broadcasted_iota(jnp.int32, xb.shape, 0) % height
    acc = None
    for kh in range(KH):
        d = kh - pad
        if d == 0:
            lhs = xb
        else:
            lhs = pltpu.roll(xb, (-d) % M, 0)
            valid = hmod < (height - d) if d > 0 else hmod >= (-d)
            lhs = jnp.where(valid, lhs, jnp.zeros_like(lhs))
        part = jnp.dot(lhs, band_ref[kh], preferred_element_type=jnp.float32)
        acc = part if acc is None else acc + part
    return acc


def _finish_bn(sacc, g, b, ch, cnt):
    tot = sacc[0:1, :]
    tsq = sacc[1:2, :]
    L = tot.shape[1]
    step = ch
    while step < L:
        tot = tot + pltpu.roll(tot, step, 1)
        tsq = tsq + pltpu.roll(tsq, step, 1)
        step *= 2
    mean = tot / cnt
    var = jnp.maximum(tsq / cnt - mean * mean, 0.0)
    scale = g * lax.rsqrt(var + _EPS)
    shift = b - mean * scale
    return scale, shift


def _accumulate(y, t, s_ref):
    part = jnp.concatenate([jnp.sum(y, 0, keepdims=True),
                            jnp.sum(y * y, 0, keepdims=True)], 0)

    @pl.when(t == 0)
    def _():
        s_ref[...] = part

    @pl.when((t > 0))
    def _():
        s_ref[...] = s_ref[...] + part


def _norm_chunk(acc_ref, s_ref, g_ref, b_ref, c_ref, j, mb, height, ch, cnt):
    scale, shift = _finish_bn(s_ref[...], g_ref[0], b_ref[0], ch, cnt)
    y = acc_ref[pl.ds(j * mb, mb), :]
    nb = mb // height
    l = y.shape[1]
    bias = jnp.broadcast_to(c_ref[0][:, None, :], (nb, height, l)).reshape(mb, l)
    return jnp.maximum(y * scale + shift, 0.0) + bias


def _k1_body(x_ref, band_ref, g_ref, b_ref, c_ref, z_ref, acc_ref, s_ref, *,
             height, t_steps, ch, cnt):
    t = pl.program_id(1)
    mb = x_ref.shape[0]

    @pl.when(t < t_steps)
    def _():
        y = _conv_taps(x_ref[...], band_ref, height)
        acc_ref[pl.ds(t * mb, mb), :] = y
        _accumulate(y, t, s_ref)

    @pl.when(t >= t_steps)
    def _():
        j = t - t_steps
        z = _norm_chunk(acc_ref, s_ref, g_ref, b_ref, c_ref, j, mb, height,
                        ch, cnt)
        z_ref[0] = z.astype(z_ref.dtype)


def _k2_body(za_ref, zb_ref, band_ref, g_ref, b_ref, c_ref, o_ref, acc_ref,
             s_ref, *, height, width, t_steps, ch, cnt):
    t = pl.program_id(1)
    mb = za_ref.shape[1]

    @pl.when(t < t_steps)
    def _():
        lhs = jnp.concatenate([za_ref[0], zb_ref[0]], axis=1)
        y = _conv_taps(lhs, band_ref, height)
        acc_ref[pl.ds(t * mb, mb), :] = y
        _accumulate(y, t, s_ref)

    @pl.when(t >= t_steps)
    def _():
        j = t - t_steps
        z = _norm_chunk(acc_ref, s_ref, g_ref, b_ref, c_ref, j, mb, height,
                        ch, cnt)
        nb = mb // height
        z = z.reshape(nb, height, width, z.shape[1] // width)
        o_ref[...] = z.transpose(0, 3, 1, 2)


def kernel(x_nchw, condition, w1_hwio, w2_hwio, bn1_gamma, bn1_beta,
           bn2_gamma, bn2_beta, emb1_w, emb1_b, emb2_w, emb2_b):
    N, Cin, H, W = x_nchw.shape
    KH, KW, _, Cout = w1_hwio.shape
    assert KH == KW and KH % 2 == 1
    assert W & (W - 1) == 0
    assert Cout % 2 == 0
    Ch = Cout // 2
    L = W * Ch
    WC_in = W * Cin
    M = N * H
    cnt = float(M * W)
    T = 16
    while (N % T) or (((N // T) * H) % 8):
        T //= 2
    Nb = N // T
    Mb = Nb * H

    x2 = jnp.transpose(x_nchw, (0, 2, 3, 1)).reshape(M, WC_in).astype(jnp.bfloat16)

    band1 = _band1_half(w1_hwio, W, jnp.bfloat16)
    band2 = _band2_half(w2_hwio, W, jnp.bfloat16)

    def half_tile(v):
        v2 = v.reshape(2, Ch)
        return jnp.tile(v2[:, None, :], (1, 1, W)).astype(jnp.float32)

    g1t, b1t = half_tile(bn1_gamma), half_tile(bn1_beta)
    g2t, b2t = half_tile(bn2_gamma), half_tile(bn2_beta)

    def half_cond(emb_w, emb_b):
        c = (condition @ emb_w.T + emb_b).reshape(N, 2, Ch)
        c = jnp.transpose(c, (1, 0, 2))
        return jnp.tile(c, (1, 1, W)).astype(jnp.float32)

    c1t = half_cond(emb1_w, emb1_b)
    c2t = half_cond(emb2_w, emb2_b)

    cparams = pltpu.CompilerParams(
        dimension_semantics=("parallel", "arbitrary"))

    def rd(i):
        return jnp.minimum(i, T - 1)

    def wr(i):
        return jnp.maximum(i - T, 0)

    z1 = pl.pallas_call(
        partial(_k1_body, height=H, t_steps=T, ch=Ch, cnt=cnt),
        grid=(2, 2 * T),
        in_specs=[
            pl.BlockSpec((Mb, WC_in), lambda k, t: (rd(t), 0)),
            pl.BlockSpec((KH, WC_in, L), lambda k, t: (0, 0, k)),
            pl.BlockSpec((1, 1, L), lambda k, t: (k, 0, 0)),
            pl.BlockSpec((1, 1, L), lambda k, t: (k, 0, 0)),
            pl.BlockSpec((1, Nb, L), lambda k, t: (k, wr(t), 0)),
        ],
        out_specs=pl.BlockSpec((1, Mb, L), lambda k, t: (k, wr(t), 0)),
        out_shape=jax.ShapeDtypeStruct((2, M, L), jnp.bfloat16),
        scratch_shapes=[pltpu.VMEM((M, L), jnp.float32),
                        pltpu.VMEM((2, L), jnp.float32)],
        compiler_params=cparams,
    )(x2, band1, g1t, b1t, c1t)

    out = pl.pallas_call(
        partial(_k2_body, height=H, width=W, t_steps=T, ch=Ch, cnt=cnt),
        grid=(2, 2 * T),
        in_specs=[
            pl.BlockSpec((1, Mb, L), lambda k, t: (0, rd(t), 0)),
            pl.BlockSpec((1, Mb, L), lambda k, t: (1, rd(t), 0)),
            pl.BlockSpec((KH, W * Cout, L), lambda k, t: (0, 0, k)),
            pl.BlockSpec((1, 1, L), lambda k, t: (k, 0, 0)),
            pl.BlockSpec((1, 1, L), lambda k, t: (k, 0, 0)),
            pl.BlockSpec((1, Nb, L), lambda k, t: (k, wr(t), 0)),
        ],
        out_specs=pl.BlockSpec((Nb, Ch, H, W), lambda k, t: (wr(t), k, 0, 0)),
        out_shape=jax.ShapeDtypeStruct((N, Cout, H, W), jnp.float32),
        scratch_shapes=[pltpu.VMEM((M, L), jnp.float32),
                        pltpu.VMEM((2, L), jnp.float32)],
        compiler_params=cparams,
    )(z1, z1, band2, g2t, b2t, c2t)

    return out


# speedup vs baseline: 65.6306x; 1.3233x over previous
import math
from functools import partial

import jax
import jax.numpy as jnp
from jax import lax
from jax.experimental import pallas as pl
from jax.experimental.pallas import tpu as pltpu

_EPS = 1e-5


def _taps(w_hwio, width):
    KH, KW, Cin, Cout = w_hwio.shape
    pad = KH // 2
    idx = jnp.arange(width)
    rel = idx[:, None] - idx[None, :] + pad
    inband = (rel >= 0) & (rel < KW)
    t = w_hwio[:, jnp.clip(rel, 0, KW - 1)]
    return jnp.where(inband[None, :, :, None, None], t, 0.0)


def _band(w_hwio, width, dtype):
    KH, KW, Cin, Cout = w_hwio.shape
    t = jnp.transpose(_taps(w_hwio, width), (0, 1, 3, 2, 4))
    return t.reshape(KH, width * Cin, width * Cout).astype(dtype)


def _band_colhalf(w_hwio, width, dtype):
    KH, KW, Cin, Cout = w_hwio.shape
    t = _taps(w_hwio, width).reshape(KH, width, width, Cin, 2, Cout // 2)
    t = jnp.transpose(t, (0, 1, 3, 4, 2, 5))
    return t.reshape(KH, width * Cin, width * Cout).astype(dtype)


def _conv_taps(xb, band_ref, height):
    M = xb.shape[0]
    KH = band_ref.shape[0]
    pad = KH // 2
    hmod = lax.broadcasted_iota(jnp.int32, xb.shape, 0) % height
    acc = None
    for kh in range(KH):
        d = kh - pad
        if d == 0:
            lhs = xb
        else:
            lhs = pltpu.roll(xb, (-d) % M, 0)
            valid = hmod < (height - d) if d > 0 else hmod >= (-d)
            lhs = jnp.where(valid, lhs, jnp.zeros_like(lhs))
        part = jnp.dot(lhs, band_ref[kh], preferred_element_type=jnp.float32)
        acc = part if acc is None else acc + part
    return acc


def _bn_act(y, g, b, period, cnt):
    tot = jnp.sum(y, axis=0, keepdims=True)
    tsq = jnp.sum(y * y, axis=0, keepdims=True)
    L = y.shape[1]
    step = period
    while step < L:
        tot = tot + pltpu.roll(tot, step, 1)
        tsq = tsq + pltpu.roll(tsq, step, 1)
        step *= 2
    mean = tot / cnt
    var = jnp.maximum(tsq / cnt - mean * mean, 0.0)
    scale = g * lax.rsqrt(var + _EPS)
    shift = b - mean * scale
    return jnp.maximum(y * scale + shift, 0.0)


def _expand_rows(c, height, m, l):
    n = m // height
    return jnp.broadcast_to(c[:, None, :], (n, height, l)).reshape(m, l)


def _mono_body(x_ref, band1_ref, band2_ref, g1_ref, b1_ref, g2_ref, b2_ref,
               c1_ref, c2_ref, o_ref, z2_ref, *, height, width, cout, ch,
               cnt, t_w):
    tw = pl.program_id(1)
    m = x_ref.shape[0]
    lh = z2_ref.shape[1]

    @pl.when(tw == 0)
    def _():
        y1 = _conv_taps(x_ref[...], band1_ref, height)
        z1 = _bn_act(y1, g1_ref[...], b1_ref[...], cout, cnt)
        z1 = z1 + _expand_rows(c1_ref[...], height, m, z1.shape[1])
        y2 = _conv_taps(z1.astype(jnp.bfloat16), band2_ref, height)
        z2 = _bn_act(y2, g2_ref[0], b2_ref[0], ch, cnt)
        z2_ref[...] = z2 + _expand_rows(c2_ref[0], height, m, lh)

    mb = m // t_w
    z = z2_ref[pl.ds(tw * mb, mb), :]
    nb = mb // height
    z = z.reshape(nb, height, width, ch)
    o_ref[...] = z.transpose(0, 3, 1, 2)


def kernel(x_nchw, condition, w1_hwio, w2_hwio, bn1_gamma, bn1_beta,
           bn2_gamma, bn2_beta, emb1_w, emb1_b, emb2_w, emb2_b):
    N, Cin, H, W = x_nchw.shape
    KH, KW, _, Cout = w1_hwio.shape
    assert KH == KW and KH % 2 == 1
    assert W & (W - 1) == 0, 'W must be a power of two for the lane all-reduce'
    assert Cout % 2 == 0
    Ch = Cout // 2
    Lh = W * Ch
    WC_in, WC = W * Cin, W * Cout
    M = N * H
    cnt = float(M * W)
    Tw = 16
    while (N % Tw) or (((N // Tw) * H) % 8):
        Tw //= 2
    Nb = N // Tw

    x2 = jnp.transpose(x_nchw, (0, 2, 3, 1)).reshape(M, WC_in).astype(jnp.bfloat16)

    band1 = _band(w1_hwio, W, jnp.bfloat16)
    band2 = _band_colhalf(w2_hwio, W, jnp.bfloat16)

    g1t = jnp.tile(bn1_gamma, W).reshape(1, WC).astype(jnp.float32)
    b1t = jnp.tile(bn1_beta, W).reshape(1, WC).astype(jnp.float32)

    def half_tile(v):
        v2 = v.reshape(2, Ch)
        return jnp.tile(v2[:, None, :], (1, 1, W)).astype(jnp.float32)

    g2t, b2t = half_tile(bn2_gamma), half_tile(bn2_beta)

    c1 = jnp.tile(condition @ emb1_w.T + emb1_b, (1, W)).astype(jnp.float32)
    c2h = (condition @ emb2_w.T + emb2_b).reshape(N, 2, Ch)
    c2h = jnp.tile(jnp.transpose(c2h, (1, 0, 2)), (1, 1, W)).astype(jnp.float32)

    out = pl.pallas_call(
        partial(_mono_body, height=H, width=W, cout=Cout, ch=Ch, cnt=cnt,
                t_w=Tw),
        grid=(2, Tw),
        in_specs=[
            pl.BlockSpec((M, WC_in), lambda k, t: (0, 0)),
            pl.BlockSpec(band1.shape, lambda k, t: (0, 0, 0)),
            pl.BlockSpec((KH, WC, Lh), lambda k, t: (0, 0, k)),
            pl.BlockSpec((1, WC), lambda k, t: (0, 0)),
            pl.BlockSpec((1, WC), lambda k, t: (0, 0)),
            pl.BlockSpec((1, 1, Lh), lambda k, t: (k, 0, 0)),
            pl.BlockSpec((1, 1, Lh), lambda k, t: (k, 0, 0)),
            pl.BlockSpec((N, WC), lambda k, t: (0, 0)),
            pl.BlockSpec((1, N, Lh), lambda k, t: (k, 0, 0)),
        ],
        out_specs=pl.BlockSpec((Nb, Ch, H, W), lambda k, t: (t, k, 0, 0)),
        out_shape=jax.ShapeDtypeStruct((N, Cout, H, W), jnp.float32),
        scratch_shapes=[pltpu.VMEM((M, Lh), jnp.float32)],
        compiler_params=pltpu.CompilerParams(
            dimension_semantics=("parallel", "arbitrary")),
    )(x2, band1, band2, g1t, b1t, g2t, b2t, c1, c2h)

    return out


# speedup vs baseline: 74.0553x; 1.1284x over previous
import math
from functools import partial

import jax
import jax.numpy as jnp
from jax import lax
from jax.experimental import pallas as pl
from jax.experimental.pallas import tpu as pltpu

_EPS = 1e-5


def _taps(w_hwio, width):
    KH, KW, Cin, Cout = w_hwio.shape
    pad = KH // 2
    idx = jnp.arange(width)
    rel = idx[:, None] - idx[None, :] + pad
    inband = (rel >= 0) & (rel < KW)
    t = w_hwio[:, jnp.clip(rel, 0, KW - 1)]
    return jnp.where(inband[None, :, :, None, None], t, 0.0)


def _band(w_hwio, width, dtype):
    KH, KW, Cin, Cout = w_hwio.shape
    t = jnp.transpose(_taps(w_hwio, width), (0, 1, 3, 2, 4))
    return t.reshape(KH, width * Cin, width * Cout).astype(dtype)


def _band_colhalf(w_hwio, width, dtype):
    KH, KW, Cin, Cout = w_hwio.shape
    t = _taps(w_hwio, width).reshape(KH, width, width, Cin, 2, Cout // 2)
    t = jnp.transpose(t, (0, 1, 3, 4, 2, 5))
    return t.reshape(KH, width * Cin, width * Cout).astype(dtype)


def _conv_taps(xb, band_ref, height):
    M = xb.shape[0]
    KH = band_ref.shape[0]
    pad = KH // 2
    hmod = lax.broadcasted_iota(jnp.int32, xb.shape, 0) % height
    acc = None
    for kh in range(KH):
        d = kh - pad
        if d == 0:
            lhs = xb
        else:
            lhs = pltpu.roll(xb, (-d) % M, 0)
            valid = hmod < (height - d) if d > 0 else hmod >= (-d)
            lhs = jnp.where(valid, lhs, jnp.zeros_like(lhs))
        part = jnp.dot(lhs, band_ref[kh], preferred_element_type=jnp.float32)
        acc = part if acc is None else acc + part
    return acc


def _bn_act(y, g, b, period, cnt):
    tot = jnp.sum(y, axis=0, keepdims=True)
    tsq = jnp.sum(y * y, axis=0, keepdims=True)
    L = y.shape[1]
    step = period
    while step < L:
        tot = tot + pltpu.roll(tot, step, 1)
        tsq = tsq + pltpu.roll(tsq, step, 1)
        step *= 2
    mean = tot / cnt
    var = jnp.maximum(tsq / cnt - mean * mean, 0.0)
    scale = g * lax.rsqrt(var + _EPS)
    shift = b - mean * scale
    return jnp.maximum(y * scale + shift, 0.0)


def _expand_rows(c, height, m, l):
    n = m // height
    return jnp.broadcast_to(c[:, None, :], (n, height, l)).reshape(m, l)


def _mono_body(x_ref, band1_ref, band2_ref, g1_ref, b1_ref, g2_ref, b2_ref,
               c1_ref, c2_ref, o_ref, *, height, cout, ch, cnt):
    m = x_ref.shape[0]
    lh = o_ref.shape[2]
    y1 = _conv_taps(x_ref[...], band1_ref, height)
    z1 = _bn_act(y1, g1_ref[...], b1_ref[...], cout, cnt)
    z1 = z1 + _expand_rows(c1_ref[...], height, m, z1.shape[1])
    y2 = _conv_taps(z1.astype(jnp.bfloat16), band2_ref, height)
    z2 = _bn_act(y2, g2_ref[0], b2_ref[0], ch, cnt)
    o_ref[0] = z2 + _expand_rows(c2_ref[0], height, m, lh)


def kernel(x_nchw, condition, w1_hwio, w2_hwio, bn1_gamma, bn1_beta,
           bn2_gamma, bn2_beta, emb1_w, emb1_b, emb2_w, emb2_b):
    N, Cin, H, W = x_nchw.shape
    KH, KW, _, Cout = w1_hwio.shape
    assert KH == KW and KH % 2 == 1
    assert W & (W - 1) == 0, 'W must be a power of two for the lane all-reduce'
    assert Cout % 2 == 0
    Ch = Cout // 2
    Lh = W * Ch
    WC_in, WC = W * Cin, W * Cout
    M = N * H
    cnt = float(M * W)

    x2 = jnp.transpose(x_nchw, (0, 2, 3, 1)).reshape(M, WC_in).astype(jnp.bfloat16)

    band1 = _band(w1_hwio, W, jnp.bfloat16)
    band2 = _band_colhalf(w2_hwio, W, jnp.bfloat16)

    g1t = jnp.tile(bn1_gamma, W).reshape(1, WC).astype(jnp.float32)
    b1t = jnp.tile(bn1_beta, W).reshape(1, WC).astype(jnp.float32)

    def half_tile(v):
        v2 = v.reshape(2, Ch)
        return jnp.tile(v2[:, None, :], (1, 1, W)).astype(jnp.float32)

    g2t, b2t = half_tile(bn2_gamma), half_tile(bn2_beta)

    c1 = jnp.tile(condition @ emb1_w.T + emb1_b, (1, W)).astype(jnp.float32)
    c2h = (condition @ emb2_w.T + emb2_b).reshape(N, 2, Ch)
    c2h = jnp.tile(jnp.transpose(c2h, (1, 0, 2)), (1, 1, W)).astype(jnp.float32)

    o2 = pl.pallas_call(
        partial(_mono_body, height=H, cout=Cout, ch=Ch, cnt=cnt),
        grid=(2,),
        in_specs=[
            pl.BlockSpec((M, WC_in), lambda k: (0, 0)),
            pl.BlockSpec(band1.shape, lambda k: (0, 0, 0)),
            pl.BlockSpec((KH, WC, Lh), lambda k: (0, 0, k)),
            pl.BlockSpec((1, WC), lambda k: (0, 0)),
            pl.BlockSpec((1, WC), lambda k: (0, 0)),
            pl.BlockSpec((1, 1, Lh), lambda k: (k, 0, 0)),
            pl.BlockSpec((1, 1, Lh), lambda k: (k, 0, 0)),
            pl.BlockSpec((N, WC), lambda k: (0, 0)),
            pl.BlockSpec((1, N, Lh), lambda k: (k, 0, 0)),
        ],
        out_specs=pl.BlockSpec((1, M, Lh), lambda k: (k, 0, 0)),
        out_shape=jax.ShapeDtypeStruct((2, M, Lh), jnp.float32),
        compiler_params=pltpu.CompilerParams(
            dimension_semantics=("parallel",)),
    )(x2, band1, band2, g1t, b1t, g2t, b2t, c1, c2h)

    out = o2.reshape(2, N, H, W, Ch)
    out = jnp.transpose(out, (1, 0, 4, 2, 3))
    return out.reshape(N, Cout, H, W)
